# Initial kernel scaffold; baseline (speedup 1.0000x reference)
import functools

import jax
import jax.numpy as jnp
from jax import lax
from jax.experimental import pallas as pl
from jax.experimental.pallas import tpu as pltpu
from jax.experimental.pallas import tpu_sc as plsc

N = 10000
E = 160000
D = 256
H = 8
DH = 32

NW = 32
EPT = E // NW
GBLK = 200
EPAD = E + 128
LBLK = 4448
NPT = 313
PBLK = 64
NEG = -1e30

_mesh = plsc.VectorSubcoreMesh(core_axis_name="c", subcore_axis_name="s")


def _wid():
  return lax.axis_index("s") * 2 + lax.axis_index("c")


@functools.partial(
    pl.kernel,
    out_type=(
        jax.ShapeDtypeStruct((EPAD, D), jnp.float32),
        jax.ShapeDtypeStruct((EPAD, D), jnp.float32),
    ),
    mesh=_mesh,
    scratch_types=[
        pltpu.VMEM((EPT,), jnp.int32),
        pltpu.VMEM((2, GBLK, D), jnp.float32),
        pltpu.SemaphoreType.DMA,
        pltpu.SemaphoreType.DMA,
    ],
)
def _gather_k(xw_hbm, src_hbm, dst_hbm, xj_hbm, xi_hbm, idx_v, buf, s0, s1):
  wid = _wid()
  base = wid * EPT
  nb = EPT // GBLK
  sems = [s0, s1]
  for which in range(2):
    iref = src_hbm if which == 0 else dst_hbm
    oref = xj_hbm if which == 0 else xi_hbm
    pltpu.sync_copy(iref.at[pl.ds(base, EPT)], idx_v)
    cps = [None, None]
    cps[0] = pltpu.async_copy(
        xw_hbm.at[idx_v.at[pl.ds(0, GBLK)]], buf.at[0], sems[0])
    for b in range(nb):
      if b + 1 < nb:
        cps[(b + 1) % 2] = pltpu.async_copy(
            xw_hbm.at[idx_v.at[pl.ds((b + 1) * GBLK, GBLK)]],
            buf.at[(b + 1) % 2], sems[(b + 1) % 2])
      cps[b % 2].wait()
      pltpu.sync_copy(buf.at[b % 2], oref.at[pl.ds(base + b * GBLK, GBLK)])


@functools.partial(
    pl.kernel,
    out_type=jax.ShapeDtypeStruct((N, D), jnp.float32),
    mesh=_mesh,
    scratch_types=[
        pltpu.VMEM((40,), jnp.int32),
        pltpu.VMEM((40 * H,), jnp.float32),
        pltpu.VMEM((2, (PBLK + 2) * H), jnp.float32),
        pltpu.VMEM((2, PBLK * D), jnp.float32),
        pltpu.VMEM((2, PBLK), jnp.int32),
        pltpu.VMEM((NPT + 7, D), jnp.float32),
        pltpu.VMEM(((NPT + 7) * H,), jnp.float32),
        pltpu.VMEM((16,), jnp.float32),
        pltpu.SemaphoreType.DMA,
        pltpu.SemaphoreType.DMA,
        pltpu.SemaphoreType.DMA,
        pltpu.SemaphoreType.DMA,
        pltpu.SemaphoreType.DMA,
        pltpu.SemaphoreType.DMA,
    ],
)
def _pull_k(e_hbm, mp_hbm, xj_hbm, dst_hbm, cuts_hbm, hout_hbm,
            cuts_v, mp_v, e_buf, xj_buf, dst_buf, ustg, dstg, tmp16,
            se0, se1, sx0, sx1, sd0, sd1):
  wid = _wid()
  node_base = wid * NPT
  lanes16 = lax.broadcasted_iota(jnp.int32, (16,), 0)
  lane8 = lax.rem(lanes16, 8)
  is_lo8 = lanes16 < 8
  zeros16 = jnp.zeros((16,), jnp.float32)

  pltpu.sync_copy(cuts_hbm, cuts_v)
  pltpu.sync_copy(mp_hbm, mp_v)

  macc = jnp.full((16,), NEG, jnp.float32)
  for r in range(0, 40, 2):
    macc = jnp.maximum(macc, mp_v[pl.ds(r * H, 16)])
  tmp16[...] = macc
  swapped = plsc.load_gather(
      tmp16, [jnp.where(is_lo8, lanes16 + 8, lanes16 - 8)])
  mfold = jnp.maximum(macc, swapped)
  tmp16[...] = mfold
  m16 = plsc.load_gather(tmp16, [lane8])

  widv = jnp.full((16,), wid, jnp.int32)
  lo = jnp.max(plsc.load_gather(cuts_v, [widv]))
  hi = jnp.max(plsc.load_gather(cuts_v, [widv + 1]))
  start8 = lo - lax.rem(lo, 8)
  npair = jnp.maximum((hi - start8 + 2 * PBLK - 1) // (2 * PBLK), 1)
  nblk = npair * 2
  lo_v = jnp.full((16,), lo, jnp.int32)
  hi_v = jnp.full((16,), hi, jnp.int32)

  def _zrow(n, _):
    for v in range(16):
      ustg[n, pl.ds(v * 16, 16)] = zeros16
    return 0
  lax.fori_loop(0, NPT + 7, _zrow, 0)

  def _zd(i, _):
    dstg[pl.ds(i * 16, 16)] = zeros16
    return 0
  lax.fori_loop(0, (NPT + 7) * H // 16, _zd, 0)

  sems_e = [se0, se1]
  sems_x = [sx0, sx1]
  sems_d = [sd0, sd1]

  def _issue(blk, slot):
    estart = start8 + blk * PBLK
    pltpu.async_copy(e_hbm.at[pl.ds(estart * H, PBLK * H)],
                     e_buf.at[slot, pl.ds(0, PBLK * H)], sems_e[slot])
    pltpu.async_copy(xj_hbm.at[pl.ds(estart * D, PBLK * D)],
                     xj_buf.at[slot], sems_x[slot])
    pltpu.async_copy(dst_hbm.at[pl.ds(estart, PBLK)],
                     dst_buf.at[slot], sems_d[slot])

  def _wait(slot):
    pltpu.make_async_copy(e_hbm.at[pl.ds(0, PBLK * H)],
                          e_buf.at[slot, pl.ds(0, PBLK * H)],
                          sems_e[slot]).wait()
    pltpu.make_async_copy(xj_hbm.at[pl.ds(0, PBLK * D)],
                          xj_buf.at[slot], sems_x[slot]).wait()
    pltpu.make_async_copy(dst_hbm.at[pl.ds(0, PBLK)],
                          dst_buf.at[slot], sems_d[slot]).wait()

  _issue(0, 0)
  _issue(1, 1)

  def _edge_body(k, carry, blk, slot):
    prev_dst, acc_d, accu = carry
    ge_v = jnp.full((16,), start8 + blk * PBLK + k, jnp.int32)
    valid = jnp.logical_and(ge_v >= lo_v, ge_v < hi_v)
    dstv = plsc.load_gather(dst_buf.at[slot],
                            [jnp.full((16,), k, jnp.int32)])
    dloc = jnp.clip(dstv - node_base, 0, NPT - 1)
    newseg = dstv != prev_dst
    e16 = e_buf[slot, pl.ds(k * H, 16)]
    p = jnp.exp(jnp.maximum(e16 - m16, -60.0))
    pm = jnp.where(valid, p, zeros16)
    acc_d = jnp.where(newseg, pm, acc_d + pm)
    plsc.store_scatter(dstg, [dloc * H + lane8], acc_d,
                       mask=jnp.logical_and(valid, is_lo8))
    tmp16[...] = pm
    pf = [plsc.load_gather(tmp16, [jnp.full((16,), h, jnp.int32)])
          for h in range(H)]
    new_accu = []
    for v in range(16):
      xjv = xj_buf[slot, pl.ds(k * D + v * 16, 16)]
      contrib = xjv * pf[v // 2]
      av = jnp.where(newseg, contrib, accu[v] + contrib)
      plsc.store_scatter(ustg, [dloc, v * 16 + lanes16], av, mask=valid)
      new_accu.append(av)
    return dstv, acc_d, new_accu

  def _blk_body(i, carry):
    for slot in range(2):
      blk = i * 2 + slot
      _wait(slot)
      def _eb(k, c, blk=blk, slot=slot):
        return _edge_body(k, c, blk, slot)
      carry = lax.fori_loop(0, PBLK, _eb, carry)
      @pl.when(blk + 2 < nblk)
      def _():
        _issue(blk + 2, slot)
    return carry

  init = (jnp.full((16,), -1, jnp.int32),
          zeros16,
          [zeros16 for _ in range(16)])
  lax.fori_loop(0, npair, _blk_body, init)

  def _norm_body(n, _):
    dvec = dstg[pl.ds(n * H, 16)]
    pos = dvec > 0.0
    invd = jnp.where(pos, 1.0 / jnp.where(pos, dvec, 1.0), zeros16)
    tmp16[...] = invd
    invf = [plsc.load_gather(tmp16, [jnp.full((16,), h, jnp.int32)])
            for h in range(H)]
    for v in range(16):
      u = ustg[n, pl.ds(v * 16, 16)]
      ustg[n, pl.ds(v * 16, 16)] = u * invf[v // 2]
    return 0
  lax.fori_loop(0, NPT, _norm_body, 0)

  nrem = N - (NW - 1) * NPT

  @pl.when(wid < NW - 1)
  def _():
    pltpu.sync_copy(ustg.at[pl.ds(0, NPT), :],
                    hout_hbm.at[pl.ds(node_base, NPT), :])

  @pl.when(wid == NW - 1)
  def _():
    pltpu.sync_copy(ustg.at[pl.ds(0, nrem), :],
                    hout_hbm.at[pl.ds(node_base, nrem), :])


def _mm_body(h_ref, w_ref, o_ref):
  o_ref[...] = jnp.dot(h_ref[...], w_ref[...],
                       preferred_element_type=jnp.float32)


def _mm_fused_body(h_ref, b_ref, w_ref, o_ref):
  t = h_ref[...] + b_ref[...]
  t = jax.nn.gelu(t, approximate=False)
  o_ref[...] = jnp.dot(t, w_ref[...], preferred_element_type=jnp.float32)


def _tc_mm(h, w):
  grid = 5
  r = N // grid
  return pl.pallas_call(
      _mm_body,
      grid=(grid,),
      in_specs=[
          pl.BlockSpec((r, D), lambda i: (i, 0)),
          pl.BlockSpec((D, D), lambda i: (0, 0)),
      ],
      out_specs=pl.BlockSpec((r, D), lambda i: (i, 0)),
      out_shape=jax.ShapeDtypeStruct((N, D), jnp.float32),
  )(h, w)


def _tc_mm_fused(h, b, w):
  grid = 5
  r = N // grid
  return pl.pallas_call(
      _mm_fused_body,
      grid=(grid,),
      in_specs=[
          pl.BlockSpec((r, D), lambda i: (i, 0)),
          pl.BlockSpec((1, D), lambda i: (0, 0)),
          pl.BlockSpec((D, D), lambda i: (0, 0)),
      ],
      out_specs=pl.BlockSpec((r, D), lambda i: (i, 0)),
      out_shape=jax.ShapeDtypeStruct((N, D), jnp.float32),
  )(h, b, w)


def _logits_body(xj_ref, xi_ref, a_ref, e_ref, mp_ref):
  i = pl.program_id(0)
  s = xj_ref[...] + xi_ref[...]
  s = jnp.where(s >= 0.0, s, 0.2 * s)
  e = jnp.dot(s, a_ref[...], preferred_element_type=jnp.float32)
  row = jax.lax.broadcasted_iota(jnp.int32, (LBLK, H), 0) + i * LBLK
  e = jnp.where(row < E, e, NEG)
  e_ref[...] = e
  mp_ref[...] = jnp.max(e, axis=0, keepdims=True)


def _tc_logits(xj, xi, a):
  grid = EPAD // LBLK
  return pl.pallas_call(
      _logits_body,
      grid=(grid,),
      in_specs=[
          pl.BlockSpec((LBLK, D), lambda i: (i, 0)),
          pl.BlockSpec((LBLK, D), lambda i: (i, 0)),
          pl.BlockSpec((D, H), lambda i: (0, 0)),
      ],
      out_specs=[
          pl.BlockSpec((LBLK, H), lambda i: (i, 0)),
          pl.BlockSpec((1, H), lambda i: (i, 0)),
      ],
      out_shape=[
          jax.ShapeDtypeStruct((EPAD, H), jnp.float32),
          jax.ShapeDtypeStruct((grid, H), jnp.float32),
      ],
  )(xj, xi, a)


def _final_body(h_ref, b3_ref, w_ref, bo_ref, g_ref, be_ref, o_ref):
  t = h_ref[...] + b3_ref[...]
  y = jnp.dot(t, w_ref[...], preferred_element_type=jnp.float32) + bo_ref[...]
  mu = jnp.mean(y, axis=-1, keepdims=True)
  var = jnp.mean((y - mu) ** 2, axis=-1, keepdims=True)
  o_ref[...] = (y - mu) * jax.lax.rsqrt(var + 1e-12) * g_ref[...] + be_ref[...]


def _tc_final(h, b3, w, bo, g, be):
  grid = 5
  r = 5000 // grid
  vec = pl.BlockSpec((1, D), lambda i: (0, 0))
  return pl.pallas_call(
      _final_body,
      grid=(grid,),
      in_specs=[
          pl.BlockSpec((r, D), lambda i: (i, 0)),
          vec,
          pl.BlockSpec((D, D), lambda i: (0, 0)),
          vec, vec, vec,
      ],
      out_specs=pl.BlockSpec((r, D), lambda i: (i, 0)),
      out_shape=jax.ShapeDtypeStruct((5000, D), jnp.float32),
  )(h, b3, w, bo, g, be)


def kernel(x, edge_index, num_trg_nodes, W0, att0, b0, W1, att1, b1,
           W2, att2, b2, W3, att3, b3, W_out, b_out, ln_gamma, ln_beta):
  src = edge_index[0]
  dst = edge_index[1]
  perm = jnp.argsort(dst)
  srcs = src[perm]
  dsts = dst[perm]
  dsts_pad = jnp.concatenate(
      [dsts, jnp.full((EPAD - E,), N, jnp.int32)])
  bounds = jnp.arange(0, NW + 1, dtype=jnp.int32) * NPT
  cuts = jnp.searchsorted(dsts, bounds).astype(jnp.int32)
  cuts = jnp.concatenate([cuts, jnp.zeros((40 - NW - 1,), jnp.int32)])

  params = [(W0, att0, b0), (W1, att1, b1), (W2, att2, b2), (W3, att3, b3)]
  cols = jnp.arange(D)
  heads = cols // DH
  amats = [jnp.zeros((D, H), jnp.float32).at[cols, heads].set(
      att[heads, cols % DH]) for _, att, _b in params]

  h = x
  prev_b = None
  for li in range(4):
    W, _att, b = params[li]
    if li == 0:
      xw = _tc_mm(h, W)
    else:
      xw = _tc_mm_fused(h, prev_b.reshape(1, D), W)
    xj, xi = _gather_k(xw, srcs, dsts)
    e, mpart = _tc_logits(xj, xi, amats[li])
    mpart_pad = jnp.concatenate(
        [mpart, jnp.full((40 - mpart.shape[0], H), NEG, jnp.float32)])
    h = _pull_k(e.reshape(-1), mpart_pad.reshape(-1), xj.reshape(-1),
                dsts_pad, cuts)
    prev_b = b

  hs = lax.dynamic_slice_in_dim(h, num_trg_nodes - 5000, 5000, axis=0)
  return _tc_final(hs, b3.reshape(1, D), W_out, b_out.reshape(1, D),
                   ln_gamma.reshape(1, D), ln_beta.reshape(1, D))


# speedup vs baseline: 15.7127x; 15.7127x over previous
import functools

import jax
import jax.numpy as jnp
from jax import lax
from jax.experimental import pallas as pl
from jax.experimental.pallas import tpu as pltpu
from jax.experimental.pallas import tpu_sc as plsc

N = 10000
E = 160000
D = 256
H = 8
DH = 32

NW = 32
EPT = E // NW
GBLK = 200
EPAD = E + 128
LBLK = 4448
NPT = 320
PBLK = 64
NEG = -1e30

_mesh = plsc.VectorSubcoreMesh(core_axis_name="c", subcore_axis_name="s")


def _wid():
  return lax.axis_index("s") * 2 + lax.axis_index("c")


@functools.partial(
    pl.kernel,
    out_type=(
        jax.ShapeDtypeStruct((EPAD, D), jnp.float32),
        jax.ShapeDtypeStruct((EPAD, D), jnp.float32),
    ),
    mesh=_mesh,
    scratch_types=[
        pltpu.VMEM((EPT,), jnp.int32),
        pltpu.VMEM((2, GBLK, D), jnp.float32),
        pltpu.SemaphoreType.DMA,
        pltpu.SemaphoreType.DMA,
    ],
    compiler_params=pltpu.CompilerParams(needs_layout_passes=False),
)
def _gather_k(xw_hbm, src_hbm, dst_hbm, xj_hbm, xi_hbm, idx_v, buf, s0, s1):
  wid = _wid()
  base = wid * EPT
  nb = EPT // GBLK
  sems = [s0, s1]
  for which in range(2):
    iref = src_hbm if which == 0 else dst_hbm
    oref = xj_hbm if which == 0 else xi_hbm
    pltpu.sync_copy(iref.at[pl.ds(base, EPT)], idx_v)
    cps = [None, None]
    cps[0] = pltpu.async_copy(
        xw_hbm.at[idx_v.at[pl.ds(0, GBLK)]], buf.at[0], sems[0])
    for b in range(nb):
      if b + 1 < nb:
        cps[(b + 1) % 2] = pltpu.async_copy(
            xw_hbm.at[idx_v.at[pl.ds((b + 1) * GBLK, GBLK)]],
            buf.at[(b + 1) % 2], sems[(b + 1) % 2])
      cps[b % 2].wait()
      pltpu.sync_copy(buf.at[b % 2], oref.at[pl.ds(base + b * GBLK, GBLK)])


@functools.partial(
    pl.kernel,
    out_type=jax.ShapeDtypeStruct((N, D), jnp.float32),
    mesh=_mesh,
    scratch_types=[
        pltpu.VMEM((40,), jnp.int32),
        pltpu.VMEM((40 * H,), jnp.float32),
        pltpu.VMEM((PBLK * H + 16,), jnp.float32),
        pltpu.VMEM((PBLK * H + 16,), jnp.float32),
        pltpu.VMEM((PBLK * D,), jnp.float32),
        pltpu.VMEM((PBLK * D,), jnp.float32),
        pltpu.VMEM((PBLK,), jnp.int32),
        pltpu.VMEM((PBLK,), jnp.int32),
        pltpu.VMEM((NPT, D), jnp.float32),
        pltpu.VMEM(((NPT + 2) * H,), jnp.float32),
        pltpu.VMEM((128,), jnp.float32),
        pltpu.SemaphoreType.DMA,
        pltpu.SemaphoreType.DMA,
        pltpu.SemaphoreType.DMA,
        pltpu.SemaphoreType.DMA,
        pltpu.SemaphoreType.DMA,
        pltpu.SemaphoreType.DMA,
    ],
    compiler_params=pltpu.CompilerParams(needs_layout_passes=False),
)
def _pull_k(e_hbm, mp_hbm, xj_hbm, dst_hbm, cuts_hbm, hout_hbm,
            cuts_v, mp_v, e_b0, e_b1, xj_b0, xj_b1, dst_b0, dst_b1,
            ustg, dstg, tmp16, se0, se1, sx0, sx1, sd0, sd1):
  wid = _wid()
  node_base = wid * NPT
  lanes16 = lax.broadcasted_iota(jnp.int32, (16,), 0)
  lane8 = lax.rem(lanes16, 8)
  is_lo8 = lanes16 < 8
  zeros16 = jnp.zeros((16,), jnp.float32)

  pltpu.sync_copy(cuts_hbm, cuts_v)
  pltpu.sync_copy(mp_hbm, mp_v)

  macc = jnp.full((16,), NEG, jnp.float32)
  for r in range(0, 40, 2):
    macc = jnp.maximum(macc, mp_v[pl.ds(r * H, 16)])
  tmp16[pl.ds(0, 16)] = macc
  swapped = plsc.load_gather(
      tmp16, [jnp.where(is_lo8, lanes16 + 8, lanes16 - 8)])
  mfold = jnp.maximum(macc, swapped)
  tmp16[pl.ds(0, 16)] = mfold
  m16 = plsc.load_gather(tmp16, [lane8])

  widv = jnp.full((16,), wid, jnp.int32)
  lo = jnp.max(plsc.load_gather(cuts_v, [widv]))
  hi = jnp.max(plsc.load_gather(cuts_v, [widv + 1]))
  start8 = lo - lax.rem(lo, 8)
  npair = jnp.maximum((hi - start8 + 2 * PBLK - 1) // (2 * PBLK), 1)
  nblk = npair * 2
  lo_v = jnp.full((16,), lo, jnp.int32)
  hi_v = jnp.full((16,), hi, jnp.int32)

  def _zrow(n, _):
    for v in range(16):
      ustg[n, pl.ds(v * 16, 16)] = zeros16
    return 0
  lax.fori_loop(0, NPT, _zrow, 0)

  def _zd(i, _):
    dstg[pl.ds(i * 16, 16)] = zeros16
    return 0
  lax.fori_loop(0, (NPT + 2) * H // 16, _zd, 0)

  sems_e = [se0, se1]
  sems_x = [sx0, sx1]
  sems_d = [sd0, sd1]
  e_bufs = [e_b0, e_b1]
  xj_bufs = [xj_b0, xj_b1]
  dst_bufs = [dst_b0, dst_b1]

  def _issue(blk, slot):
    estart = pl.multiple_of(start8 + blk * PBLK, 8)
    pltpu.async_copy(e_hbm.at[pl.ds(estart * H, PBLK * H + 16)],
                     e_bufs[slot], sems_e[slot])
    pltpu.async_copy(xj_hbm.at[pl.ds(estart * D, PBLK * D)],
                     xj_bufs[slot], sems_x[slot])
    pltpu.async_copy(dst_hbm.at[pl.ds(estart, PBLK)],
                     dst_bufs[slot], sems_d[slot])

  def _wait(slot):
    pltpu.make_async_copy(e_hbm.at[pl.ds(0, PBLK * H + 16)],
                          e_bufs[slot], sems_e[slot]).wait()
    pltpu.make_async_copy(xj_hbm.at[pl.ds(0, PBLK * D)],
                          xj_bufs[slot], sems_x[slot]).wait()
    pltpu.make_async_copy(dst_hbm.at[pl.ds(0, PBLK)],
                          dst_bufs[slot], sems_d[slot]).wait()

  _issue(0, 0)
  _issue(1, 1)

  def _edge_body(k, carry, blk, slot):
    prev_dst, acc_d, accu = carry
    ge_v = jnp.full((16,), start8 + blk * PBLK + k, jnp.int32)
    valid = jnp.logical_and(ge_v >= lo_v, ge_v < hi_v)
    dstv = plsc.load_gather(dst_bufs[slot],
                            [jnp.full((16,), k, jnp.int32)])
    dloc = jnp.clip(dstv - node_base, 0, NPT - 1)
    newseg = dstv != prev_dst
    e16 = e_bufs[slot][pl.ds(k * H, 16)]
    p = jnp.exp(jnp.maximum(e16 - m16, -60.0))
    pm = jnp.where(valid, p, zeros16)
    acc_d = jnp.where(newseg, pm, acc_d + pm)
    plsc.store_scatter(dstg, [dloc * H + lane8], acc_d,
                       mask=jnp.logical_and(valid, is_lo8))
    tmp16[pl.ds(8, 16)] = pm
    pf = [plsc.load_gather(tmp16, [jnp.full((16,), 8 + h, jnp.int32)])
          for h in range(H)]
    new_accu = []
    for v in range(16):
      xjv = xj_bufs[slot][pl.ds(k * D + v * 16, 16)]
      contrib = xjv * pf[v // 2]
      av = jnp.where(newseg, contrib, accu[v] + contrib)
      plsc.store_scatter(ustg, [dloc, v * 16 + lanes16], av, mask=valid)
      new_accu.append(av)
    return dstv, acc_d, new_accu

  def _blk_body(i, carry):
    for slot in range(2):
      blk = i * 2 + slot
      _wait(slot)
      def _eb(k, c, blk=blk, slot=slot):
        return _edge_body(k, c, blk, slot)
      carry = lax.fori_loop(0, PBLK, _eb, carry)
      @pl.when(blk + 2 < nblk)
      def _():
        _issue(blk + 2, slot)
    return carry

  init = (jnp.full((16,), -1, jnp.int32),
          zeros16,
          [zeros16 for _ in range(16)])
  lax.fori_loop(0, npair, _blk_body, init)

  def _norm_body(n, _):
    dvec = dstg[pl.ds(n * H, 16)]
    pos = dvec > 0.0
    invd = jnp.where(pos, 1.0 / jnp.where(pos, dvec, 1.0), zeros16)
    tmp16[pl.ds(8, 16)] = invd
    invf = [plsc.load_gather(tmp16, [jnp.full((16,), 8 + h, jnp.int32)])
            for h in range(H)]
    for v in range(16):
      u = ustg[n, pl.ds(v * 16, 16)]
      ustg[n, pl.ds(v * 16, 16)] = u * invf[v // 2]
    return 0
  lax.fori_loop(0, NPT, _norm_body, 0)

  nrem = N - (NW - 1) * NPT

  @pl.when(wid < NW - 1)
  def _():
    pltpu.sync_copy(ustg.at[pl.ds(0, NPT), :],
                    hout_hbm.at[pl.ds(node_base, NPT), :])

  @pl.when(wid == NW - 1)
  def _():
    pltpu.sync_copy(ustg.at[pl.ds(0, nrem), :],
                    hout_hbm.at[pl.ds(node_base, nrem), :])


def _mm_body(h_ref, w_ref, o_ref):
  o_ref[...] = jnp.dot(h_ref[...], w_ref[...],
                       preferred_element_type=jnp.float32)


def _mm_fused_body(h_ref, b_ref, w_ref, o_ref):
  t = h_ref[...] + b_ref[...]
  t = 0.5 * t * (1.0 + lax.erf(t * 0.7071067811865476))
  o_ref[...] = jnp.dot(t, w_ref[...], preferred_element_type=jnp.float32)


def _tc_mm(h, w):
  grid = 5
  r = N // grid
  return pl.pallas_call(
      _mm_body,
      grid=(grid,),
      in_specs=[
          pl.BlockSpec((r, D), lambda i: (i, 0)),
          pl.BlockSpec((D, D), lambda i: (0, 0)),
      ],
      out_specs=pl.BlockSpec((r, D), lambda i: (i, 0)),
      out_shape=jax.ShapeDtypeStruct((N, D), jnp.float32),
  )(h, w)


def _tc_mm_fused(h, b, w):
  grid = 5
  r = N // grid
  return pl.pallas_call(
      _mm_fused_body,
      grid=(grid,),
      in_specs=[
          pl.BlockSpec((r, D), lambda i: (i, 0)),
          pl.BlockSpec((1, D), lambda i: (0, 0)),
          pl.BlockSpec((D, D), lambda i: (0, 0)),
      ],
      out_specs=pl.BlockSpec((r, D), lambda i: (i, 0)),
      out_shape=jax.ShapeDtypeStruct((N, D), jnp.float32),
  )(h, b, w)


def _logits_body(xj_ref, xi_ref, a_ref, e_ref, mp_ref):
  i = pl.program_id(0)
  s = xj_ref[...] + xi_ref[...]
  s = jnp.where(s >= 0.0, s, 0.2 * s)
  e = jnp.dot(s, a_ref[...], preferred_element_type=jnp.float32)
  row = jax.lax.broadcasted_iota(jnp.int32, (LBLK, H), 0) + i * LBLK
  e = jnp.where(row < E, e, NEG)
  e_ref[...] = e
  mp_ref[...] = jnp.max(e, axis=0, keepdims=True).reshape(1, 1, H)


def _tc_logits(xj, xi, a):
  grid = EPAD // LBLK
  return pl.pallas_call(
      _logits_body,
      grid=(grid,),
      in_specs=[
          pl.BlockSpec((LBLK, D), lambda i: (i, 0)),
          pl.BlockSpec((LBLK, D), lambda i: (i, 0)),
          pl.BlockSpec((D, H), lambda i: (0, 0)),
      ],
      out_specs=[
          pl.BlockSpec((LBLK, H), lambda i: (i, 0)),
          pl.BlockSpec((1, 1, H), lambda i: (i, 0, 0)),
      ],
      out_shape=[
          jax.ShapeDtypeStruct((EPAD, H), jnp.float32),
          jax.ShapeDtypeStruct((grid, 1, H), jnp.float32),
      ],
  )(xj, xi, a)


def _final_body(h_ref, b3_ref, w_ref, bo_ref, g_ref, be_ref, o_ref):
  t = h_ref[...] + b3_ref[...]
  y = jnp.dot(t, w_ref[...], preferred_element_type=jnp.float32) + bo_ref[...]
  mu = jnp.mean(y, axis=-1, keepdims=True)
  var = jnp.mean((y - mu) ** 2, axis=-1, keepdims=True)
  o_ref[...] = (y - mu) * jax.lax.rsqrt(var + 1e-12) * g_ref[...] + be_ref[...]


def _tc_final(h, b3, w, bo, g, be):
  grid = 5
  r = 5000 // grid
  vec = pl.BlockSpec((1, D), lambda i: (0, 0))
  return pl.pallas_call(
      _final_body,
      grid=(grid,),
      in_specs=[
          pl.BlockSpec((r, D), lambda i: (i, 0)),
          vec,
          pl.BlockSpec((D, D), lambda i: (0, 0)),
          vec, vec, vec,
      ],
      out_specs=pl.BlockSpec((r, D), lambda i: (i, 0)),
      out_shape=jax.ShapeDtypeStruct((5000, D), jnp.float32),
  )(h, b3, w, bo, g, be)


def kernel(x, edge_index, num_trg_nodes, W0, att0, b0, W1, att1, b1,
           W2, att2, b2, W3, att3, b3, W_out, b_out, ln_gamma, ln_beta):
  src = edge_index[0]
  dst = edge_index[1]
  perm = jnp.argsort(dst)
  srcs = src[perm]
  dsts = dst[perm]
  dsts_pad = jnp.concatenate(
      [dsts, jnp.full((EPAD - E,), N, jnp.int32)])
  bounds = jnp.arange(0, NW + 1, dtype=jnp.int32) * NPT
  cuts = jnp.searchsorted(dsts, bounds).astype(jnp.int32)
  cuts = jnp.concatenate([cuts, jnp.zeros((40 - NW - 1,), jnp.int32)])

  params = [(W0, att0, b0), (W1, att1, b1), (W2, att2, b2), (W3, att3, b3)]
  cols = jnp.arange(D)
  heads = cols // DH
  amats = [jnp.zeros((D, H), jnp.float32).at[cols, heads].set(
      att[heads, cols % DH]) for _, att, _b in params]

  h = x
  prev_b = None
  for li in range(4):
    W, _att, b = params[li]
    if li == 0:
      xw = _tc_mm(h, W)
    else:
      xw = _tc_mm_fused(h, prev_b.reshape(1, D), W)
    xj, xi = _gather_k(xw, srcs, dsts)
    e, mpart = _tc_logits(xj, xi, amats[li])
    mpart = mpart.reshape(-1, H)
    mpart_pad = jnp.concatenate(
        [mpart, jnp.full((40 - mpart.shape[0], H), NEG, jnp.float32)])
    h = _pull_k(e.reshape(-1), mpart_pad.reshape(-1), xj.reshape(-1),
                dsts_pad, cuts)
    prev_b = b

  hs = lax.dynamic_slice_in_dim(h, num_trg_nodes - 5000, 5000, axis=0)
  return _tc_final(hs, b3.reshape(1, D), W_out, b_out.reshape(1, D),
                   ln_gamma.reshape(1, D), ln_beta.reshape(1, D))


# speedup vs baseline: 15.7187x; 1.0004x over previous
import functools

import jax
import jax.numpy as jnp
from jax import lax
from jax.experimental import pallas as pl
from jax.experimental.pallas import tpu as pltpu
from jax.experimental.pallas import tpu_sc as plsc

N = 10000
E = 160000
D = 256
H = 8
DH = 32

NW = 32
EPT = E // NW
GBLK = 200
EPAD = E + 128
LBLK = 4448
NPT = 320
PBLK = 64
NEG = -1e30

_mesh = plsc.VectorSubcoreMesh(core_axis_name="c", subcore_axis_name="s")


def _wid():
  return lax.axis_index("s") * 2 + lax.axis_index("c")


@functools.partial(
    pl.kernel,
    out_type=(
        jax.ShapeDtypeStruct((EPAD, D), jnp.float32),
        jax.ShapeDtypeStruct((EPAD, D), jnp.float32),
    ),
    mesh=_mesh,
    scratch_types=[
        pltpu.VMEM((EPT,), jnp.int32),
        pltpu.VMEM((2, GBLK, D), jnp.float32),
        pltpu.SemaphoreType.DMA,
        pltpu.SemaphoreType.DMA,
    ],
    compiler_params=pltpu.CompilerParams(needs_layout_passes=False),
)
def _gather_k(xw_hbm, src_hbm, dst_hbm, xj_hbm, xi_hbm, idx_v, buf, s0, s1):
  wid = _wid()
  base = wid * EPT
  nb = EPT // GBLK
  sems = [s0, s1]
  for which in range(2):
    iref = src_hbm if which == 0 else dst_hbm
    oref = xj_hbm if which == 0 else xi_hbm
    pltpu.sync_copy(iref.at[pl.ds(base, EPT)], idx_v)
    cps = [None, None]
    cps[0] = pltpu.async_copy(
        xw_hbm.at[idx_v.at[pl.ds(0, GBLK)]], buf.at[0], sems[0])
    for b in range(nb):
      if b + 1 < nb:
        cps[(b + 1) % 2] = pltpu.async_copy(
            xw_hbm.at[idx_v.at[pl.ds((b + 1) * GBLK, GBLK)]],
            buf.at[(b + 1) % 2], sems[(b + 1) % 2])
      cps[b % 2].wait()
      pltpu.sync_copy(buf.at[b % 2], oref.at[pl.ds(base + b * GBLK, GBLK)])


@functools.partial(
    pl.kernel,
    out_type=jax.ShapeDtypeStruct((N, D), jnp.float32),
    mesh=_mesh,
    scratch_types=[
        pltpu.VMEM((40,), jnp.int32),
        pltpu.VMEM((40 * H,), jnp.float32),
        pltpu.VMEM((PBLK * H + 16,), jnp.float32),
        pltpu.VMEM((PBLK * H + 16,), jnp.float32),
        pltpu.VMEM((PBLK * D,), jnp.float32),
        pltpu.VMEM((PBLK * D,), jnp.float32),
        pltpu.VMEM((PBLK,), jnp.int32),
        pltpu.VMEM((PBLK,), jnp.int32),
        pltpu.VMEM((NPT, D), jnp.float32),
        pltpu.VMEM(((NPT + 2) * H,), jnp.float32),
        pltpu.VMEM((128,), jnp.float32),
        pltpu.SemaphoreType.DMA,
        pltpu.SemaphoreType.DMA,
        pltpu.SemaphoreType.DMA,
        pltpu.SemaphoreType.DMA,
        pltpu.SemaphoreType.DMA,
        pltpu.SemaphoreType.DMA,
    ],
    compiler_params=pltpu.CompilerParams(needs_layout_passes=False),
)
def _pull_k(e_hbm, mp_hbm, xj_hbm, dst_hbm, cuts_hbm, hout_hbm,
            cuts_v, mp_v, e_b0, e_b1, xj_b0, xj_b1, dst_b0, dst_b1,
            ustg, dstg, tmp16, se0, se1, sx0, sx1, sd0, sd1):
  wid = _wid()
  node_base = wid * NPT
  lanes16 = lax.broadcasted_iota(jnp.int32, (16,), 0)
  lane8 = lax.rem(lanes16, 8)
  is_lo8 = lanes16 < 8
  zeros16 = jnp.zeros((16,), jnp.float32)

  pltpu.sync_copy(cuts_hbm, cuts_v)
  pltpu.sync_copy(mp_hbm, mp_v)

  macc = jnp.full((16,), NEG, jnp.float32)
  for r in range(0, 40, 2):
    macc = jnp.maximum(macc, mp_v[pl.ds(r * H, 16)])
  tmp16[pl.ds(0, 16)] = macc
  swapped = plsc.load_gather(
      tmp16, [jnp.where(is_lo8, lanes16 + 8, lanes16 - 8)])
  mfold = jnp.maximum(macc, swapped)
  tmp16[pl.ds(0, 16)] = mfold
  m16 = plsc.load_gather(tmp16, [lane8])

  widv = jnp.full((16,), wid, jnp.int32)
  lo = jnp.max(plsc.load_gather(cuts_v, [widv]))
  hi = jnp.max(plsc.load_gather(cuts_v, [widv + 1]))
  start8 = lo - lax.rem(lo, 8)
  npair = jnp.maximum((hi - start8 + 2 * PBLK - 1) // (2 * PBLK), 1)
  nblk = npair * 2
  lo_v = jnp.full((16,), lo, jnp.int32)
  hi_v = jnp.full((16,), hi, jnp.int32)

  def _zrow(n, _):
    for v in range(16):
      ustg[n, pl.ds(v * 16, 16)] = zeros16
    return 0
  lax.fori_loop(0, NPT, _zrow, 0)

  def _zd(i, _):
    dstg[pl.ds(i * 16, 16)] = zeros16
    return 0
  lax.fori_loop(0, (NPT + 2) * H // 16, _zd, 0)

  sems_e = [se0, se1]
  sems_x = [sx0, sx1]
  sems_d = [sd0, sd1]
  e_bufs = [e_b0, e_b1]
  xj_bufs = [xj_b0, xj_b1]
  dst_bufs = [dst_b0, dst_b1]

  def _issue(blk, slot):
    estart = pl.multiple_of(start8 + blk * PBLK, 8)
    pltpu.async_copy(e_hbm.at[pl.ds(estart * H, PBLK * H + 16)],
                     e_bufs[slot], sems_e[slot])
    pltpu.async_copy(xj_hbm.at[pl.ds(estart * D, PBLK * D)],
                     xj_bufs[slot], sems_x[slot])
    pltpu.async_copy(dst_hbm.at[pl.ds(estart, PBLK)],
                     dst_bufs[slot], sems_d[slot])

  def _wait(slot):
    pltpu.make_async_copy(e_hbm.at[pl.ds(0, PBLK * H + 16)],
                          e_bufs[slot], sems_e[slot]).wait()
    pltpu.make_async_copy(xj_hbm.at[pl.ds(0, PBLK * D)],
                          xj_bufs[slot], sems_x[slot]).wait()
    pltpu.make_async_copy(dst_hbm.at[pl.ds(0, PBLK)],
                          dst_bufs[slot], sems_d[slot]).wait()

  _issue(0, 0)
  _issue(1, 1)

  def _edge_body(k, carry, blk, slot):
    prev_dst, acc_d, accu = carry
    ge_v = jnp.full((16,), start8 + blk * PBLK + k, jnp.int32)
    valid = jnp.logical_and(ge_v >= lo_v, ge_v < hi_v)
    dstv = plsc.load_gather(dst_bufs[slot],
                            [jnp.full((16,), k, jnp.int32)])
    dloc = jnp.clip(dstv - node_base, 0, NPT - 1)
    newseg = dstv != prev_dst
    e16 = e_bufs[slot][pl.ds(k * H, 16)]
    p = jnp.exp(jnp.maximum(e16 - m16, -60.0))
    pm = jnp.where(valid, p, zeros16)
    acc_d = jnp.where(newseg, pm, acc_d + pm)
    plsc.store_scatter(dstg, [dloc * H + lane8], acc_d,
                       mask=jnp.logical_and(valid, is_lo8))
    tmp16[pl.ds(8, 16)] = pm
    pf = [plsc.load_gather(tmp16, [jnp.full((16,), 8 + h, jnp.int32)])
          for h in range(H)]
    new_accu = []
    for v in range(16):
      xjv = xj_bufs[slot][pl.ds(k * D + v * 16, 16)]
      contrib = xjv * pf[v // 2]
      av = jnp.where(newseg, contrib, accu[v] + contrib)
      plsc.store_scatter(ustg, [dloc, v * 16 + lanes16], av, mask=valid)
      new_accu.append(av)
    return dstv, acc_d, new_accu

  def _blk_body(i, carry):
    for slot in range(2):
      blk = i * 2 + slot
      _wait(slot)
      def _eb(k, c, blk=blk, slot=slot):
        return _edge_body(k, c, blk, slot)
      carry = lax.fori_loop(0, PBLK, _eb, carry, unroll=4)
      @pl.when(blk + 2 < nblk)
      def _():
        _issue(blk + 2, slot)
    return carry

  init = (jnp.full((16,), -1, jnp.int32),
          zeros16,
          [zeros16 for _ in range(16)])
  lax.fori_loop(0, npair, _blk_body, init)

  def _norm_body(n, _):
    dvec = dstg[pl.ds(n * H, 16)]
    pos = dvec > 0.0
    invd = jnp.where(pos, 1.0 / jnp.where(pos, dvec, 1.0), zeros16)
    tmp16[pl.ds(8, 16)] = invd
    invf = [plsc.load_gather(tmp16, [jnp.full((16,), 8 + h, jnp.int32)])
            for h in range(H)]
    for v in range(16):
      u = ustg[n, pl.ds(v * 16, 16)]
      ustg[n, pl.ds(v * 16, 16)] = u * invf[v // 2]
    return 0
  lax.fori_loop(0, NPT, _norm_body, 0)

  nrem = N - (NW - 1) * NPT

  @pl.when(wid < NW - 1)
  def _():
    pltpu.sync_copy(ustg.at[pl.ds(0, NPT), :],
                    hout_hbm.at[pl.ds(node_base, NPT), :])

  @pl.when(wid == NW - 1)
  def _():
    pltpu.sync_copy(ustg.at[pl.ds(0, nrem), :],
                    hout_hbm.at[pl.ds(node_base, nrem), :])


def _mm_body(h_ref, w_ref, o_ref):
  o_ref[...] = jnp.dot(h_ref[...], w_ref[...],
                       preferred_element_type=jnp.float32)


def _mm_fused_body(h_ref, b_ref, w_ref, o_ref):
  t = h_ref[...] + b_ref[...]
  t = 0.5 * t * (1.0 + lax.erf(t * 0.7071067811865476))
  o_ref[...] = jnp.dot(t, w_ref[...], preferred_element_type=jnp.float32)


def _tc_mm(h, w):
  grid = 5
  r = N // grid
  return pl.pallas_call(
      _mm_body,
      grid=(grid,),
      in_specs=[
          pl.BlockSpec((r, D), lambda i: (i, 0)),
          pl.BlockSpec((D, D), lambda i: (0, 0)),
      ],
      out_specs=pl.BlockSpec((r, D), lambda i: (i, 0)),
      out_shape=jax.ShapeDtypeStruct((N, D), jnp.float32),
  )(h, w)


def _tc_mm_fused(h, b, w):
  grid = 5
  r = N // grid
  return pl.pallas_call(
      _mm_fused_body,
      grid=(grid,),
      in_specs=[
          pl.BlockSpec((r, D), lambda i: (i, 0)),
          pl.BlockSpec((1, D), lambda i: (0, 0)),
          pl.BlockSpec((D, D), lambda i: (0, 0)),
      ],
      out_specs=pl.BlockSpec((r, D), lambda i: (i, 0)),
      out_shape=jax.ShapeDtypeStruct((N, D), jnp.float32),
  )(h, b, w)


def _logits_body(xj_ref, xi_ref, a_ref, e_ref, mp_ref):
  i = pl.program_id(0)
  s = xj_ref[...] + xi_ref[...]
  s = jnp.where(s >= 0.0, s, 0.2 * s)
  e = jnp.dot(s, a_ref[...], preferred_element_type=jnp.float32)
  row = jax.lax.broadcasted_iota(jnp.int32, (LBLK, H), 0) + i * LBLK
  e = jnp.where(row < E, e, NEG)
  e_ref[...] = e
  mp_ref[...] = jnp.max(e, axis=0, keepdims=True).reshape(1, 1, H)


def _tc_logits(xj, xi, a):
  grid = EPAD // LBLK
  return pl.pallas_call(
      _logits_body,
      grid=(grid,),
      in_specs=[
          pl.BlockSpec((LBLK, D), lambda i: (i, 0)),
          pl.BlockSpec((LBLK, D), lambda i: (i, 0)),
          pl.BlockSpec((D, H), lambda i: (0, 0)),
      ],
      out_specs=[
          pl.BlockSpec((LBLK, H), lambda i: (i, 0)),
          pl.BlockSpec((1, 1, H), lambda i: (i, 0, 0)),
      ],
      out_shape=[
          jax.ShapeDtypeStruct((EPAD, H), jnp.float32),
          jax.ShapeDtypeStruct((grid, 1, H), jnp.float32),
      ],
  )(xj, xi, a)


def _final_body(h_ref, b3_ref, w_ref, bo_ref, g_ref, be_ref, o_ref):
  t = h_ref[...] + b3_ref[...]
  y = jnp.dot(t, w_ref[...], preferred_element_type=jnp.float32) + bo_ref[...]
  mu = jnp.mean(y, axis=-1, keepdims=True)
  var = jnp.mean((y - mu) ** 2, axis=-1, keepdims=True)
  o_ref[...] = (y - mu) * jax.lax.rsqrt(var + 1e-12) * g_ref[...] + be_ref[...]


def _tc_final(h, b3, w, bo, g, be):
  grid = 5
  r = 5000 // grid
  vec = pl.BlockSpec((1, D), lambda i: (0, 0))
  return pl.pallas_call(
      _final_body,
      grid=(grid,),
      in_specs=[
          pl.BlockSpec((r, D), lambda i: (i, 0)),
          vec,
          pl.BlockSpec((D, D), lambda i: (0, 0)),
          vec, vec, vec,
      ],
      out_specs=pl.BlockSpec((r, D), lambda i: (i, 0)),
      out_shape=jax.ShapeDtypeStruct((5000, D), jnp.float32),
  )(h, b3, w, bo, g, be)


def kernel(x, edge_index, num_trg_nodes, W0, att0, b0, W1, att1, b1,
           W2, att2, b2, W3, att3, b3, W_out, b_out, ln_gamma, ln_beta):
  src = edge_index[0]
  dst = edge_index[1]
  perm = jnp.argsort(dst)
  srcs = src[perm]
  dsts = dst[perm]
  dsts_pad = jnp.concatenate(
      [dsts, jnp.full((EPAD - E,), N, jnp.int32)])
  bounds = jnp.arange(0, NW + 1, dtype=jnp.int32) * NPT
  cuts = jnp.searchsorted(dsts, bounds).astype(jnp.int32)
  cuts = jnp.concatenate([cuts, jnp.zeros((40 - NW - 1,), jnp.int32)])

  params = [(W0, att0, b0), (W1, att1, b1), (W2, att2, b2), (W3, att3, b3)]
  cols = jnp.arange(D)
  heads = cols // DH
  amats = [jnp.zeros((D, H), jnp.float32).at[cols, heads].set(
      att[heads, cols % DH]) for _, att, _b in params]

  h = x
  prev_b = None
  for li in range(4):
    W, _att, b = params[li]
    if li == 0:
      xw = _tc_mm(h, W)
    else:
      xw = _tc_mm_fused(h, prev_b.reshape(1, D), W)
    xj, xi = _gather_k(xw, srcs, dsts)
    e, mpart = _tc_logits(xj, xi, amats[li])
    mpart = mpart.reshape(-1, H)
    mpart_pad = jnp.concatenate(
        [mpart, jnp.full((40 - mpart.shape[0], H), NEG, jnp.float32)])
    h = _pull_k(e.reshape(-1), mpart_pad.reshape(-1), xj.reshape(-1),
                dsts_pad, cuts)
    prev_b = b

  hs = lax.dynamic_slice_in_dim(h, num_trg_nodes - 5000, 5000, axis=0)
  return _tc_final(hs, b3.reshape(1, D), W_out, b_out.reshape(1, D),
                   ln_gamma.reshape(1, D), ln_beta.reshape(1, D))
